# baseline (speedup 1.0000x reference)
import jax
import jax.numpy as jnp
from jax import lax
from jax.experimental import pallas as pl
from jax.experimental.pallas import tpu as pltpu
from jax.experimental.pallas import tpu_sc as plsc

NC = 2
NS = 16
NW = NC * NS
L = 16
B = 4096
D = 512
DP = D + L
ND = 8
NM = 2 * ND
RPW = B // NW
NCH = D // L
EPS = 1e-3


def _stats_body(x_hbm, idx_hbm, parts_hbm, rows_v, idx_v, part_v, sem):
    c = lax.axis_index("c")
    s = lax.axis_index("s")
    wid = s * NC + c
    base = wid * RPW

    cp_rows = pltpu.async_copy(x_hbm.at[pl.ds(base, RPW)], rows_v, sem)
    pltpu.sync_copy(idx_hbm.at[pl.ds(base, RPW)], idx_v.at[pl.ds(0, RPW)])

    zeros = jnp.zeros((L,), jnp.float32)
    for d in range(NM):
        for ch in range(NCH + 1):
            part_v[d, pl.ds(ch * L, L)] = zeros

    cp_rows.wait()
    ones = jnp.full((L,), 1.0, jnp.float32)

    @plsc.parallel_loop(0, RPW, step=1, unroll=4)
    def row_body(r):
        d = idx_v[pl.ds(r, L)][0]
        for ch in range(NCH):
            sl = pl.ds(ch * L, L)
            x = rows_v[r, sl]
            plsc.addupdate(part_v.at[d, sl], x)
            plsc.addupdate(part_v.at[d + ND, sl], x * x)
        plsc.addupdate(part_v.at[d, pl.ds(D, L)], ones)

    pltpu.sync_copy(part_v, parts_hbm.at[wid])


def _finalize_body(parts_ref, gg_ref, gb_ref, dg_ref, db_ref,
                   scale_ref, shift_ref):
    p = parts_ref[...]
    sums = jnp.sum(p[:, :ND, :D], axis=0)
    sqs = jnp.sum(p[:, ND:, :D], axis=0)
    cnt = jnp.sum(p[:, :ND, D], axis=0)
    n = jnp.maximum(cnt, 1.0)[:, None]
    mean = sums / n
    var = sqs / n - mean * mean
    scale = (gg_ref[...][None, :] + dg_ref[...]) * lax.rsqrt(var + EPS)
    shift = gb_ref[...][None, :] + db_ref[...] - scale * mean
    scale_ref[...] = scale
    shift_ref[...] = shift


LSTR = RPW + 4


def _apply_body(x_hbm, idx_hbm, scale_hbm, shift_hbm, out_hbm,
                rows_v, idx_v, scale_v, shift_v, lists_s, cnt_s, sem):
    c = lax.axis_index("c")
    s = lax.axis_index("s")
    wid = s * NC + c
    base = wid * RPW

    cp_rows = pltpu.async_copy(x_hbm.at[pl.ds(base, RPW)],
                               rows_v.at[pl.ds(0, RPW)], sem)
    pltpu.sync_copy(idx_hbm.at[pl.ds(base, RPW)], idx_v.at[pl.ds(0, RPW)])
    pltpu.sync_copy(scale_hbm, scale_v)
    pltpu.sync_copy(shift_hbm, shift_v)

    for d in range(ND):
        cnt_s[d] = 0

    def bucket_body(r, carry):
        d = idx_v[pl.ds(r, L)][0]
        k = cnt_s[d]
        lists_s[d * LSTR + k] = r
        cnt_s[d] = k + 1
        return carry

    lax.fori_loop(0, RPW, bucket_body, 0)
    for d in range(ND):
        k = cnt_s[d]
        lists_s[d * LSTR + k] = RPW
        lists_s[d * LSTR + k + 1] = RPW
        lists_s[d * LSTR + k + 2] = RPW

    cp_rows.wait()

    for d in range(ND):
        trips = (cnt_s[d] + 3) >> 2

        def ch_body(ch, carry, d=d, trips=trips):
            sl = pl.ds(ch * L, L)
            sreg = scale_v[d, sl]
            hreg = shift_v[d, sl]

            def quad_body(i, carry2):
                i4 = i * 4
                for j in range(4):
                    r = lists_s[d * LSTR + i4 + j]
                    rows_v[r, sl] = rows_v[r, sl] * sreg + hreg
                return carry2

            lax.fori_loop(0, trips, quad_body, 0)
            return carry

        lax.fori_loop(0, NCH, ch_body, 0)

    pltpu.sync_copy(rows_v.at[pl.ds(0, RPW)], out_hbm.at[pl.ds(base, RPW)])


def _mesh():
    return plsc.VectorSubcoreMesh(core_axis_name="c", subcore_axis_name="s",
                                  num_cores=NC, num_subcores=NS)


def kernel(inputs, global_gamma, global_beta, domain_gamma, domain_beta,
           domain_index):
    idx32 = domain_index.astype(jnp.int32)

    parts = pl.kernel(
        _stats_body,
        out_type=jax.ShapeDtypeStruct((NW, NM, DP), jnp.float32),
        mesh=_mesh(),
        scratch_types=[
            pltpu.VMEM((RPW, D), jnp.float32),
            pltpu.VMEM((RPW + L,), jnp.int32),
            pltpu.VMEM((NM, DP), jnp.float32),
            pltpu.SemaphoreType.DMA,
        ],
    )(inputs, idx32)

    scale, shift = pl.pallas_call(
        _finalize_body,
        out_shape=(
            jax.ShapeDtypeStruct((ND, D), jnp.float32),
            jax.ShapeDtypeStruct((ND, D), jnp.float32),
        ),
    )(parts, global_gamma, global_beta, domain_gamma, domain_beta)

    out = pl.kernel(
        _apply_body,
        out_type=jax.ShapeDtypeStruct((B, D), jnp.float32),
        mesh=_mesh(),
        scratch_types=[
            pltpu.VMEM((RPW + 1, D), jnp.float32),
            pltpu.VMEM((RPW + L,), jnp.int32),
            pltpu.VMEM((ND, D), jnp.float32),
            pltpu.VMEM((ND, D), jnp.float32),
            pltpu.SMEM((ND * LSTR,), jnp.int32),
            pltpu.SMEM((ND,), jnp.int32),
            pltpu.SemaphoreType.DMA,
        ],
    )(inputs, idx32, scale, shift)

    return out


# speedup vs baseline: 1.4296x; 1.4296x over previous
import jax
import jax.numpy as jnp
from jax import lax
from jax.experimental import pallas as pl
from jax.experimental.pallas import tpu as pltpu
from jax.experimental.pallas import tpu_sc as plsc

NC = 2
NS = 16
NW = NC * NS
L = 16
B = 4096
D = 512
DP = D + L
ND = 8
NM = 2 * ND
RPW = B // NW
NCH = D // L
EPS = 1e-3


def _stats_body(x_hbm, idx_hbm, parts_hbm, rows_v, idx_v, part_v, sem):
    c = lax.axis_index("c")
    s = lax.axis_index("s")
    wid = s * NC + c
    base = wid * RPW

    cp_rows = pltpu.async_copy(x_hbm.at[pl.ds(base, RPW)], rows_v, sem)
    pltpu.sync_copy(idx_hbm.at[pl.ds(base, RPW)], idx_v.at[pl.ds(0, RPW)])

    zeros = jnp.zeros((L,), jnp.float32)
    for d in range(NM):
        for ch in range(NCH + 1):
            part_v[d, pl.ds(ch * L, L)] = zeros

    cp_rows.wait()
    ones = jnp.full((L,), 1.0, jnp.float32)

    @plsc.parallel_loop(0, RPW, step=1, unroll=4)
    def row_body(r):
        d = idx_v[pl.ds(r, L)][0]
        for ch in range(NCH):
            sl = pl.ds(ch * L, L)
            x = rows_v[r, sl]
            plsc.addupdate(part_v.at[d, sl], x)
            plsc.addupdate(part_v.at[d + ND, sl], x * x)
        plsc.addupdate(part_v.at[d, pl.ds(D, L)], ones)

    pltpu.sync_copy(part_v, parts_hbm.at[wid])


def _finalize_body(parts_ref, gg_ref, gb_ref, dg_ref, db_ref,
                   scale_ref, shift_ref):
    p = parts_ref[...]
    sums = jnp.sum(p[:, :ND, :D], axis=0)
    sqs = jnp.sum(p[:, ND:, :D], axis=0)
    cnt = jnp.sum(p[:, :ND, D], axis=0)
    n = jnp.maximum(cnt, 1.0)[:, None]
    mean = sums / n
    var = sqs / n - mean * mean
    scale = (gg_ref[...][None, :] + dg_ref[...]) * lax.rsqrt(var + EPS)
    shift = gb_ref[...][None, :] + db_ref[...] - scale * mean
    scale_ref[...] = scale
    shift_ref[...] = shift


def _apply_body(x_hbm, idx_hbm, scale_hbm, shift_hbm, out_hbm,
                rows_v, idx_v, scale_v, shift_v, sem):
    c = lax.axis_index("c")
    s = lax.axis_index("s")
    wid = s * NC + c
    base = wid * RPW

    cp_rows = pltpu.async_copy(x_hbm.at[pl.ds(base, RPW)], rows_v, sem)
    pltpu.sync_copy(idx_hbm.at[pl.ds(base, RPW)], idx_v.at[pl.ds(0, RPW)])
    pltpu.sync_copy(scale_hbm, scale_v)
    pltpu.sync_copy(shift_hbm, shift_v)
    cp_rows.wait()

    @plsc.parallel_loop(0, RPW, step=1, unroll=4)
    def row_body(r):
        d = idx_v[pl.ds(r, L)][0]
        for ch in range(NCH):
            sl = pl.ds(ch * L, L)
            rows_v[r, sl] = rows_v[r, sl] * scale_v[d, sl] + shift_v[d, sl]

    pltpu.sync_copy(rows_v, out_hbm.at[pl.ds(base, RPW)])


def _mesh():
    return plsc.VectorSubcoreMesh(core_axis_name="c", subcore_axis_name="s",
                                  num_cores=NC, num_subcores=NS)


def kernel(inputs, global_gamma, global_beta, domain_gamma, domain_beta,
           domain_index):
    idx32 = domain_index.astype(jnp.int32)

    parts = pl.kernel(
        _stats_body,
        out_type=jax.ShapeDtypeStruct((NW, NM, DP), jnp.float32),
        mesh=_mesh(),
        scratch_types=[
            pltpu.VMEM((RPW, D), jnp.float32),
            pltpu.VMEM((RPW + L,), jnp.int32),
            pltpu.VMEM((NM, DP), jnp.float32),
            pltpu.SemaphoreType.DMA,
        ],
    )(inputs, idx32)

    scale, shift = pl.pallas_call(
        _finalize_body,
        out_shape=(
            jax.ShapeDtypeStruct((ND, D), jnp.float32),
            jax.ShapeDtypeStruct((ND, D), jnp.float32),
        ),
    )(parts, global_gamma, global_beta, domain_gamma, domain_beta)

    out = pl.kernel(
        _apply_body,
        out_type=jax.ShapeDtypeStruct((B, D), jnp.float32),
        mesh=_mesh(),
        scratch_types=[
            pltpu.VMEM((RPW, D), jnp.float32),
            pltpu.VMEM((RPW + L,), jnp.int32),
            pltpu.VMEM((ND, D), jnp.float32),
            pltpu.VMEM((ND, D), jnp.float32),
            pltpu.SemaphoreType.DMA,
        ],
    )(inputs, idx32, scale, shift)

    return out
